# Initial kernel scaffold; baseline (speedup 1.0000x reference)
import jax
import jax.numpy as jnp
from jax.experimental import pallas as pl


def kernel(inputs, indices):
    raise NotImplementedError("write your pallas kernel here")


# speedup vs baseline: 3.4558x; 3.4558x over previous
import functools

import jax
import jax.numpy as jnp
from jax import lax
from jax.experimental import pallas as pl
from jax.experimental.pallas import tpu as pltpu
from jax.experimental.pallas import tpu_sc as plsc

_N_NODES = 10000
_SPIRAL = 32
_C = 128

_NW = 32
_B = _N_NODES * _SPIRAL
_BPW = _B // _NW
_CHUNK = 128
_NFULL = _BPW // _CHUNK
_TAIL = _BPW - _NFULL * _CHUNK


def _make_gather():
    mesh = plsc.VectorSubcoreMesh(core_axis_name="c", subcore_axis_name="s")

    @functools.partial(
        pl.kernel,
        mesh=mesh,
        out_type=jax.ShapeDtypeStruct((_B, _C), jnp.float32),
        scratch_types=[
            pltpu.VMEM((_BPW,), jnp.int32),
            pltpu.VMEM((_CHUNK, _C), jnp.float32),
            pltpu.VMEM((_CHUNK, _C), jnp.float32),
            pltpu.SemaphoreType.DMA,
            pltpu.SemaphoreType.DMA,
        ],
    )
    def gather_rows(table_hbm, idx_hbm, out_hbm, idx_v, buf0, buf1, sem0, sem1):
        wid = lax.axis_index("s") * 2 + lax.axis_index("c")
        base = wid * _BPW
        pltpu.sync_copy(idx_hbm.at[pl.ds(base, _BPW)], idx_v)

        def chunk_body(j, carry):
            start = pl.multiple_of(j * _CHUNK, _CHUNK)
            cp = pltpu.async_copy(
                table_hbm.at[idx_v.at[pl.ds(start, _CHUNK)]], buf0, sem0)
            cp.wait()
            pltpu.sync_copy(buf0, out_hbm.at[pl.ds(base + start, _CHUNK)])
            return carry

        lax.fori_loop(0, _NFULL, chunk_body, 0)

        tail_start = _NFULL * _CHUNK
        cp = pltpu.async_copy(
            table_hbm.at[idx_v.at[pl.ds(tail_start, _TAIL)]],
            buf1.at[pl.ds(0, _TAIL)], sem1)
        cp.wait()
        pltpu.sync_copy(buf1.at[pl.ds(0, _TAIL)],
                        out_hbm.at[pl.ds(base + tail_start, _TAIL)])

    return gather_rows


_GATHER = _make_gather()


def kernel(inputs, indices):
    table = inputs.reshape(_N_NODES, _C)
    flat_idx = indices.reshape(-1).astype(jnp.int32)
    out = _GATHER(table, flat_idx)
    return out.reshape(1, _N_NODES, _SPIRAL * _C)


# speedup vs baseline: 4.0631x; 1.1757x over previous
import functools

import jax
import jax.numpy as jnp
from jax import lax
from jax.experimental import pallas as pl
from jax.experimental.pallas import tpu as pltpu
from jax.experimental.pallas import tpu_sc as plsc

_N_NODES = 10000
_SPIRAL = 32
_C = 128

_NW = 32
_B = _N_NODES * _SPIRAL
_BPW = _B // _NW
_CHUNK = 128
_NFULL = _BPW // _CHUNK
_TAIL = _BPW - _NFULL * _CHUNK


def _make_gather():
    mesh = plsc.VectorSubcoreMesh(core_axis_name="c", subcore_axis_name="s")

    @functools.partial(
        pl.kernel,
        mesh=mesh,
        out_type=jax.ShapeDtypeStruct((_B, _C), jnp.float32),
        scratch_types=[
            pltpu.VMEM((_BPW,), jnp.int32),
            pltpu.VMEM((_CHUNK, _C), jnp.float32),
            pltpu.VMEM((_CHUNK, _C), jnp.float32),
            pltpu.SemaphoreType.DMA,
            pltpu.SemaphoreType.DMA,
        ],
    )
    def gather_rows(table_hbm, idx_hbm, out_hbm, idx_v, buf0, buf1, sem0, sem1):
        wid = lax.axis_index("s") * 2 + lax.axis_index("c")
        base = wid * _BPW
        pltpu.sync_copy(idx_hbm.at[pl.ds(base, _BPW)], idx_v)

        def start_gather(j, buf, sem):
            start = pl.multiple_of(j * _CHUNK, _CHUNK)
            pltpu.async_copy(
                table_hbm.at[idx_v.at[pl.ds(start, _CHUNK)]], buf, sem)

        def wait_gather(buf, sem):
            pltpu.make_async_copy(
                table_hbm.at[pl.ds(0, _CHUNK)], buf, sem).wait()

        def store(j, buf):
            start = pl.multiple_of(j * _CHUNK, _CHUNK)
            pltpu.sync_copy(buf, out_hbm.at[pl.ds(base + start, _CHUNK)])

        start_gather(0, buf0, sem0)

        def pair_body(j2, carry):
            j = j2 * 2
            start_gather(j + 1, buf1, sem1)
            wait_gather(buf0, sem0)
            store(j, buf0)
            start_gather(j + 2, buf0, sem0)
            wait_gather(buf1, sem1)
            store(j + 1, buf1)
            return carry

        lax.fori_loop(0, _NFULL // 2 - 1, pair_body, 0)

        j = _NFULL - 2
        start_gather(j + 1, buf1, sem1)
        wait_gather(buf0, sem0)
        store(j, buf0)
        tail_start = _NFULL * _CHUNK
        pltpu.async_copy(
            table_hbm.at[idx_v.at[pl.ds(tail_start, _TAIL)]],
            buf0.at[pl.ds(0, _TAIL)], sem0)
        wait_gather(buf1, sem1)
        store(j + 1, buf1)
        pltpu.make_async_copy(
            table_hbm.at[pl.ds(0, _TAIL)], buf0.at[pl.ds(0, _TAIL)],
            sem0).wait()
        pltpu.sync_copy(buf0.at[pl.ds(0, _TAIL)],
                        out_hbm.at[pl.ds(base + tail_start, _TAIL)])

    return gather_rows


_GATHER = _make_gather()


def kernel(inputs, indices):
    table = inputs.reshape(_N_NODES, _C)
    flat_idx = indices.reshape(-1).astype(jnp.int32)
    out = _GATHER(table, flat_idx)
    return out.reshape(1, _N_NODES, _SPIRAL * _C)


# speedup vs baseline: 4.0964x; 1.0082x over previous
import functools

import jax
import jax.numpy as jnp
from jax import lax
from jax.experimental import pallas as pl
from jax.experimental.pallas import tpu as pltpu
from jax.experimental.pallas import tpu_sc as plsc

_N_NODES = 10000
_SPIRAL = 32
_C = 128

_NW = 32
_B = _N_NODES * _SPIRAL
_BPW = _B // _NW
_CHUNK = 128
_NFULL = _BPW // _CHUNK
_TAIL = _BPW - _NFULL * _CHUNK


_NBUF = 6
_LEAD = 3


def _make_gather():
    mesh = plsc.VectorSubcoreMesh(core_axis_name="c", subcore_axis_name="s")

    scratch = (
        [pltpu.VMEM((_BPW,), jnp.int32)]
        + [pltpu.VMEM((_CHUNK, _C), jnp.float32) for _ in range(_NBUF)]
        + [pltpu.VMEM((_TAIL, _C), jnp.float32)]
        + [pltpu.SemaphoreType.DMA for _ in range(2 * _NBUF + 1)]
    )

    @functools.partial(
        pl.kernel,
        mesh=mesh,
        out_type=jax.ShapeDtypeStruct((_B, _C), jnp.float32),
        scratch_types=scratch,
    )
    def gather_rows(table_hbm, idx_hbm, out_hbm, idx_v, *refs):
        bufs = refs[:_NBUF]
        tbuf = refs[_NBUF]
        gsem = refs[_NBUF + 1:2 * _NBUF + 1]
        ssem = refs[2 * _NBUF + 1:3 * _NBUF + 1]
        tsem = refs[3 * _NBUF + 1]

        wid = lax.axis_index("s") * 2 + lax.axis_index("c")
        base = wid * _BPW
        pltpu.sync_copy(idx_hbm.at[pl.ds(base, _BPW)], idx_v)

        def start_gather(j, b):
            start = pl.multiple_of(j * _CHUNK, _CHUNK)
            pltpu.async_copy(
                table_hbm.at[idx_v.at[pl.ds(start, _CHUNK)]], bufs[b],
                gsem[b])

        def wait_gather(b):
            pltpu.make_async_copy(
                table_hbm.at[pl.ds(0, _CHUNK)], bufs[b], gsem[b]).wait()

        def start_store(j, b):
            start = pl.multiple_of(j * _CHUNK, _CHUNK)
            pltpu.async_copy(
                bufs[b], out_hbm.at[pl.ds(base + start, _CHUNK)], ssem[b])

        def wait_store(b):
            pltpu.make_async_copy(
                table_hbm.at[pl.ds(0, _CHUNK)], bufs[b], ssem[b]).wait()

        tail_start = _NFULL * _CHUNK

        for j in range(_LEAD):
            start_gather(j, j % _NBUF)
        pltpu.async_copy(
            table_hbm.at[idx_v.at[pl.ds(tail_start, _TAIL)]], tbuf, tsem)
        for j in range(_LEAD):
            wait_gather(j % _NBUF)
            start_store(j, j % _NBUF)
            start_gather(j + _LEAD, (j + _LEAD) % _NBUF)

        def group_body(g, carry):
            jg = _LEAD + g * _NBUF
            for b in range(_NBUF):
                j = jg + b
                slot = (_LEAD + b) % _NBUF
                wait_gather(slot)
                start_store(j, slot)
                wait_store(b)
                start_gather(j + _LEAD, b)
            return carry

        n_groups = (_NFULL - 2 * _LEAD) // _NBUF
        lax.fori_loop(0, n_groups, group_body, 0)

        for j in range(_NFULL - _LEAD, _NFULL):
            slot = j % _NBUF
            wait_gather(slot)
            start_store(j, slot)
            wait_store((j + _LEAD) % _NBUF)
        for j in range(_NFULL - _LEAD, _NFULL):
            wait_store(j % _NBUF)
        pltpu.make_async_copy(
            table_hbm.at[pl.ds(0, _TAIL)], tbuf, tsem).wait()
        pltpu.sync_copy(tbuf, out_hbm.at[pl.ds(base + tail_start, _TAIL)])

    return gather_rows


_GATHER = _make_gather()


def kernel(inputs, indices):
    table = inputs.reshape(_N_NODES, _C)
    flat_idx = indices.reshape(-1).astype(jnp.int32)
    out = _GATHER(table, flat_idx)
    return out.reshape(1, _N_NODES, _SPIRAL * _C)


# speedup vs baseline: 7.6057x; 1.8567x over previous
import functools

import jax
import jax.numpy as jnp
from jax import lax
from jax.experimental import pallas as pl
from jax.experimental.pallas import tpu as pltpu
from jax.experimental.pallas import tpu_sc as plsc

_N_NODES = 10000
_SPIRAL = 32
_C = 128

_NW = 32
_B = _N_NODES * _SPIRAL
_BPW = _B // _NW
_CHUNK = 128
_NFULL = _BPW // _CHUNK
_TAIL = _BPW - _NFULL * _CHUNK


_NBUF = 6
_LEAD = 3


def _make_gather():
    mesh = plsc.VectorSubcoreMesh(core_axis_name="c", subcore_axis_name="s")

    scratch = (
        [pltpu.VMEM((_BPW,), jnp.int32)]
        + [pltpu.VMEM((_CHUNK, _C), jnp.float32) for _ in range(_NBUF)]
        + [pltpu.VMEM((_TAIL, _C), jnp.float32)]
        + [pltpu.SemaphoreType.DMA for _ in range(2 * _NBUF + 1)]
    )

    @functools.partial(
        pl.kernel,
        mesh=mesh,
        out_type=jax.ShapeDtypeStruct((_B, _C), jnp.float32),
        scratch_types=scratch,
    )
    def gather_rows(table_hbm, idx_hbm, out_hbm, idx_v, *refs):
        bufs = refs[:_NBUF]
        tbuf = refs[_NBUF]
        gsem = refs[_NBUF + 1:2 * _NBUF + 1]
        ssem = refs[2 * _NBUF + 1:3 * _NBUF + 1]
        tsem = refs[3 * _NBUF + 1]

        wid = lax.axis_index("s") * 2 + lax.axis_index("c")
        base = wid * _BPW
        pltpu.sync_copy(idx_hbm.at[pl.ds(base, _BPW)], idx_v)

        def start_gather(j, b):
            start = pl.multiple_of(j * _CHUNK, _CHUNK)
            pltpu.async_copy(
                table_hbm.at[idx_v.at[pl.ds(start, _CHUNK)]], bufs[b],
                gsem[b])

        def wait_gather(b):
            pltpu.make_async_copy(
                table_hbm.at[pl.ds(0, _CHUNK)], bufs[b], gsem[b]).wait()

        def start_store(j, b):
            start = pl.multiple_of(j * _CHUNK, _CHUNK)
            pltpu.async_copy(
                bufs[b], out_hbm.at[pl.ds(base + start, _CHUNK)], ssem[b])

        def wait_store(b):
            pltpu.make_async_copy(
                table_hbm.at[pl.ds(0, _CHUNK)], bufs[b], ssem[b]).wait()

        tail_start = _NFULL * _CHUNK

        for j in range(_LEAD):
            start_gather(j, j % _NBUF)
        pltpu.async_copy(
            table_hbm.at[idx_v.at[pl.ds(tail_start, _TAIL)]], tbuf, tsem)
        for j in range(_LEAD):
            wait_gather(j % _NBUF)
            start_store(j, j % _NBUF)
            start_gather(j + _LEAD, (j + _LEAD) % _NBUF)

        def group_body(g, carry):
            jg = _LEAD + g * _NBUF
            for b in range(_NBUF):
                j = jg + b
                slot = (_LEAD + b) % _NBUF
                wait_gather(slot)
                start_store(j, slot)
                wait_store(b)
                start_gather(j + _LEAD, b)
            return carry

        n_groups = (_NFULL - 2 * _LEAD) // _NBUF
        lax.fori_loop(0, n_groups, group_body, 0)

        for j in range(_NFULL - _LEAD, _NFULL):
            slot = j % _NBUF
            wait_gather(slot)
            start_store(j, slot)
            wait_store((j + _LEAD) % _NBUF)
        for j in range(_NFULL - _LEAD, _NFULL):
            wait_store(j % _NBUF)
        pltpu.make_async_copy(
            table_hbm.at[pl.ds(0, _TAIL)], tbuf, tsem).wait()
        pltpu.sync_copy(tbuf, out_hbm.at[pl.ds(base + tail_start, _TAIL)])

    return gather_rows


_GATHER = _make_gather()


def kernel(inputs, indices):
    table = inputs.reshape(_N_NODES, _C)
    n8 = _N_NODES // 8
    idx_perm = (indices.astype(jnp.int32)
                .reshape(n8, 8, _SPIRAL)
                .transpose(0, 2, 1)
                .reshape(-1))
    out = _GATHER(table, idx_perm)
    return (out.reshape(n8, _SPIRAL, 8, _C)
            .transpose(0, 2, 1, 3)
            .reshape(1, _N_NODES, _SPIRAL * _C))


# speedup vs baseline: 10.3677x; 1.3632x over previous
import functools

import jax
import jax.numpy as jnp
from jax import lax
from jax.experimental import pallas as pl
from jax.experimental.pallas import tpu as pltpu
from jax.experimental.pallas import tpu_sc as plsc

_N_NODES = 10000
_SPIRAL = 32
_C = 128

_NW = 32
_B = _N_NODES * _SPIRAL
_BPW = _B // _NW
_CHUNK = 128
_NFULL = _BPW // _CHUNK
_TAIL = _BPW - _NFULL * _CHUNK


_NBUF = 2
_LEAD = 1


def _make_gather():
    mesh = plsc.VectorSubcoreMesh(core_axis_name="c", subcore_axis_name="s")

    scratch = (
        [pltpu.VMEM((_BPW,), jnp.int32)]
        + [pltpu.VMEM((_CHUNK, _C), jnp.float32) for _ in range(_NBUF)]
        + [pltpu.VMEM((_TAIL, _C), jnp.float32)]
        + [pltpu.SemaphoreType.DMA for _ in range(2 * _NBUF + 1)]
        + [pltpu.VMEM_SHARED((_N_NODES, _C), jnp.float32),
           pltpu.SemaphoreType.DMA]
    )

    @functools.partial(
        pl.kernel,
        mesh=mesh,
        out_type=jax.ShapeDtypeStruct((_B, _C), jnp.float32),
        scratch_types=scratch,
    )
    def gather_rows(table_hbm, idx_hbm, out_hbm, idx_v, *refs):
        bufs = refs[:_NBUF]
        tbuf = refs[_NBUF]
        gsem = refs[_NBUF + 1:2 * _NBUF + 1]
        ssem = refs[2 * _NBUF + 1:3 * _NBUF + 1]
        tsem = refs[3 * _NBUF + 1]
        table_spm = refs[3 * _NBUF + 2]
        stage_sem = refs[3 * _NBUF + 3]

        sid = lax.axis_index("s")
        wid = sid * 2 + lax.axis_index("c")
        base = wid * _BPW

        @pl.when(sid == 0)
        def _():
            pltpu.async_copy(table_hbm, table_spm, stage_sem).wait()

        pltpu.sync_copy(idx_hbm.at[pl.ds(base, _BPW)], idx_v)
        plsc.subcore_barrier()

        def start_gather(j, b):
            start = pl.multiple_of(j * _CHUNK, _CHUNK)
            pltpu.async_copy(
                table_spm.at[idx_v.at[pl.ds(start, _CHUNK)]], bufs[b],
                gsem[b])

        def wait_gather(b):
            pltpu.make_async_copy(
                table_hbm.at[pl.ds(0, _CHUNK)], bufs[b], gsem[b]).wait()

        def start_store(j, b):
            start = pl.multiple_of(j * _CHUNK, _CHUNK)
            pltpu.async_copy(
                bufs[b], out_hbm.at[pl.ds(base + start, _CHUNK)], ssem[b])

        def wait_store(b):
            pltpu.make_async_copy(
                table_hbm.at[pl.ds(0, _CHUNK)], bufs[b], ssem[b]).wait()

        tail_start = _NFULL * _CHUNK

        for j in range(_LEAD):
            start_gather(j, j % _NBUF)
        pltpu.async_copy(
            table_spm.at[idx_v.at[pl.ds(tail_start, _TAIL)]], tbuf, tsem)
        for j in range(_LEAD):
            wait_gather(j % _NBUF)
            start_store(j, j % _NBUF)
            start_gather(j + _LEAD, (j + _LEAD) % _NBUF)

        def group_body(g, carry):
            jg = _LEAD + g * _NBUF
            for b in range(_NBUF):
                j = jg + b
                slot = (_LEAD + b) % _NBUF
                wait_gather(slot)
                start_store(j, slot)
                wait_store(b)
                start_gather(j + _LEAD, b)
            return carry

        n_groups = (_NFULL - 2 * _LEAD) // _NBUF
        lax.fori_loop(0, n_groups, group_body, 0)

        for j in range(_NFULL - _LEAD, _NFULL):
            slot = j % _NBUF
            wait_gather(slot)
            start_store(j, slot)
            wait_store((j + _LEAD) % _NBUF)
        for j in range(_NFULL - _LEAD, _NFULL):
            wait_store(j % _NBUF)
        pltpu.make_async_copy(
            table_hbm.at[pl.ds(0, _TAIL)], tbuf, tsem).wait()
        pltpu.sync_copy(tbuf, out_hbm.at[pl.ds(base + tail_start, _TAIL)])

    return gather_rows


_GATHER = _make_gather()


def kernel(inputs, indices):
    table = inputs.reshape(_N_NODES, _C)
    n8 = _N_NODES // 8
    idx_perm = (indices.astype(jnp.int32)
                .reshape(n8, 8, _SPIRAL)
                .transpose(0, 2, 1)
                .reshape(-1))
    out = _GATHER(table, idx_perm)
    return (out.reshape(n8, _SPIRAL, 8, _C)
            .transpose(0, 2, 1, 3)
            .reshape(1, _N_NODES, _SPIRAL * _C))


# speedup vs baseline: 12.8812x; 1.2424x over previous
import functools

import jax
import jax.numpy as jnp
from jax import lax
from jax.experimental import pallas as pl
from jax.experimental.pallas import tpu as pltpu
from jax.experimental.pallas import tpu_sc as plsc

_N_NODES = 10000
_SPIRAL = 32
_C = 128

_NW = 32
_NPW_HI = 313
_NPW_LO = 312
_CHUNK_NODES = 4
_CHUNK = _CHUNK_NODES * _SPIRAL
_NFULL = _NPW_LO // _CHUNK_NODES
_N8 = _N_NODES // 8

_NBUF = 2
_LEAD = 1


def _make_gather():
    mesh = plsc.VectorSubcoreMesh(core_axis_name="c", subcore_axis_name="s")

    scratch = (
        [pltpu.VMEM((_NPW_HI * _SPIRAL,), jnp.int32)]
        + [pltpu.VMEM((_CHUNK, _C), jnp.float32) for _ in range(_NBUF)]
        + [pltpu.VMEM((_SPIRAL, _C), jnp.float32)]
        + [pltpu.SemaphoreType.DMA for _ in range(2 * _NBUF + 1)]
        + [pltpu.VMEM_SHARED((_N_NODES, _C), jnp.float32),
           pltpu.SemaphoreType.DMA]
    )

    @functools.partial(
        pl.kernel,
        mesh=mesh,
        out_type=jax.ShapeDtypeStruct((_N8, _SPIRAL, 8, _C), jnp.float32),
        scratch_types=scratch,
    )
    def gather_rows(table_hbm, idx_hbm, out_hbm, idx_v, *refs):
        bufs = refs[:_NBUF]
        tbuf = refs[_NBUF]
        gsem = refs[_NBUF + 1:2 * _NBUF + 1]
        ssem = refs[2 * _NBUF + 1:3 * _NBUF + 1]
        tsem = refs[3 * _NBUF + 1]
        table_spm = refs[3 * _NBUF + 2]
        stage_sem = refs[3 * _NBUF + 3]

        sid = lax.axis_index("s")
        wid = sid * 2 + lax.axis_index("c")
        nstart = wid * _NPW_LO + jnp.maximum(wid - _NW // 2, 0)
        base = nstart * _SPIRAL

        @pl.when(sid == 0)
        def _():
            pltpu.async_copy(table_hbm, table_spm, stage_sem).wait()

        pltpu.sync_copy(
            idx_hbm.at[pl.ds(base, _NPW_HI * _SPIRAL)], idx_v)
        plsc.subcore_barrier()

        def start_gather(j, b):
            start = pl.multiple_of(j * _CHUNK, _CHUNK)
            pltpu.async_copy(
                table_spm.at[idx_v.at[pl.ds(start, _CHUNK)]], bufs[b],
                gsem[b])

        def wait_gather(b):
            pltpu.make_async_copy(
                table_hbm.at[pl.ds(0, _CHUNK)], bufs[b], gsem[b]).wait()

        def start_store(j, b):
            for q in range(_CHUNK_NODES):
                n = nstart + j * _CHUNK_NODES + q
                pltpu.async_copy(
                    bufs[b].at[pl.ds(q * _SPIRAL, _SPIRAL)],
                    out_hbm.at[n // 8, :, n % 8, :],
                    ssem[b])

        def wait_store(b):
            pltpu.make_async_copy(
                table_hbm.at[pl.ds(0, _CHUNK)], bufs[b], ssem[b]).wait()

        for j in range(_LEAD):
            start_gather(j, j % _NBUF)

        @pl.when(wid >= _NW // 2)
        def _():
            start = pl.multiple_of(_NFULL * _CHUNK, _SPIRAL)
            pltpu.async_copy(
                table_spm.at[idx_v.at[pl.ds(start, _SPIRAL)]], tbuf, tsem)

        for j in range(_LEAD):
            wait_gather(j % _NBUF)
            start_store(j, j % _NBUF)
            start_gather(j + _LEAD, (j + _LEAD) % _NBUF)

        def group_body(g, carry):
            jg = _LEAD + g * _NBUF
            for b in range(_NBUF):
                j = jg + b
                slot = (_LEAD + b) % _NBUF
                wait_gather(slot)
                start_store(j, slot)
                wait_store(b)
                start_gather(j + _LEAD, b)
            return carry

        n_groups = (_NFULL - 2 * _LEAD) // _NBUF
        lax.fori_loop(0, n_groups, group_body, 0)

        for j in range(_NFULL - _LEAD, _NFULL):
            slot = j % _NBUF
            wait_gather(slot)
            start_store(j, slot)
            wait_store((j + _LEAD) % _NBUF)
        for j in range(_NFULL - _LEAD, _NFULL):
            wait_store(j % _NBUF)

        @pl.when(wid >= _NW // 2)
        def _():
            pltpu.make_async_copy(
                table_hbm.at[pl.ds(0, _SPIRAL)], tbuf, tsem).wait()
            n = nstart + _NFULL * _CHUNK_NODES
            pltpu.sync_copy(tbuf, out_hbm.at[n // 8, :, n % 8, :])

    return gather_rows


_GATHER = _make_gather()


def kernel(inputs, indices):
    table = inputs.reshape(_N_NODES, _C)
    idx_flat = indices.astype(jnp.int32).reshape(-1)
    out = _GATHER(table, idx_flat)
    return (out.transpose(0, 2, 1, 3)
            .reshape(1, _N_NODES, _SPIRAL * _C))
